# baseline (speedup 1.0000x reference)
import functools

import jax
import jax.numpy as jnp
from jax import lax
from jax.experimental import pallas as pl
from jax.experimental.pallas import tpu as pltpu
from jax.experimental.pallas import tpu_sc as plsc

_D = 1024
_B = 4
_S = 2048
_NW = 32
_SPW = _S // _NW
_RB = 4
_NCH = _SPW // _RB
_JW = 8
_JB = _D // (16 * _JW)


def _sc_body(x_hbm, codes_hbm, w_hbm, abs_hbm, out_hbm,
             inx0, inx1, ina0, ina1, ou0, ou1, w_v, codes_v,
             isem0, isem1, osem0, osem1):
    wid = lax.axis_index("s") * 2 + lax.axis_index("c")
    s_base = wid * _SPW
    inxs = (inx0, inx1)
    inas = (ina0, ina1)
    ous = (ou0, ou1)
    isems = (isem0, isem1)
    osems = (osem0, osem1)

    pltpu.sync_copy(w_hbm, w_v)
    for bb in range(_B):
        pltpu.sync_copy(codes_hbm.at[pl.ds(bb * _S + s_base, _SPW)],
                        codes_v.at[pl.ds(bb * _SPW, _SPW)])

    def start_in(m, i):
        for bb in range(_B):
            pltpu.async_copy(
                x_hbm.at[pl.ds(bb * _S + s_base + m * _RB, _RB)],
                inxs[i].at[pl.ds(bb * _RB, _RB)], isems[i])
        pltpu.async_copy(abs_hbm.at[pl.ds(s_base + m * _RB, _RB)],
                         inas[i], isems[i])

    def drain_in(i):
        for bb in range(_B):
            pltpu.make_async_copy(
                x_hbm.at[pl.ds(0, _RB)],
                inxs[i].at[pl.ds(0, _RB)], isems[i]).wait()
        pltpu.make_async_copy(abs_hbm.at[pl.ds(0, _RB)], inas[i],
                              isems[i]).wait()

    def start_out(m, i):
        for bb in range(_B):
            pltpu.async_copy(
                ous[i].at[pl.ds(bb * _RB, _RB)],
                out_hbm.at[pl.ds(bb * _S + s_base + m * _RB, _RB)], osems[i])

    def drain_out(i):
        for bb in range(_B):
            pltpu.make_async_copy(
                ous[i].at[pl.ds(bb * _RB, _RB)],
                out_hbm.at[pl.ds(0, _RB)], osems[i]).wait()

    for i in range(2):
        start_in(i, i)

    def step(k, _):
        for i in range(2):
            m = 2 * k + i
            drain_in(i)

            @pl.when(k > 0)
            def _():
                drain_out(i)

            @plsc.parallel_loop(0, _RB, 1, unroll=2)
            def _(t, i=i, m=m):
                cs = []
                for bb in range(_B):
                    cv = codes_v[pl.ds(bb * _SPW + m * _RB + t, 16)]
                    cs.append(cv[0])
                for jb in range(_JB):
                    aregs = []
                    for jj in range(_JW):
                        aregs.append(
                            inas[i][t, pl.ds(jb * _JW * 16 + jj * 16, 16)])
                    for bb in range(_B):
                        r = bb * _RB + t
                        for jj in range(_JW):
                            sl = pl.ds(jb * _JW * 16 + jj * 16, 16)
                            ous[i][r, sl] = (inxs[i][r, sl] + aregs[jj]
                                             + w_v[cs[bb], sl])

            start_out(m, i)

            @pl.when(k < (_NCH // 2 - 1))
            def _():
                start_in(m + 2, i)
        return 0

    lax.fori_loop(0, _NCH // 2, step, 0)
    for i in range(2):
        drain_out(i)


def kernel(x, pos_codes, struct_w, abs_emb):
    b, s, d = x.shape
    x2 = x.reshape(b * s, d)
    codes = pos_codes.astype(jnp.int32).reshape(b * s)
    mesh = plsc.VectorSubcoreMesh(core_axis_name="c", subcore_axis_name="s")
    run = functools.partial(
        pl.kernel,
        mesh=mesh,
        out_type=jax.ShapeDtypeStruct((b * s, d), jnp.float32),
        scratch_types=[
            pltpu.VMEM((_B * _RB, _D), jnp.float32),
            pltpu.VMEM((_B * _RB, _D), jnp.float32),
            pltpu.VMEM((_RB, _D), jnp.float32),
            pltpu.VMEM((_RB, _D), jnp.float32),
            pltpu.VMEM((_B * _RB, _D), jnp.float32),
            pltpu.VMEM((_B * _RB, _D), jnp.float32),
            pltpu.VMEM((5, _D), jnp.float32),
            pltpu.VMEM((_B * _SPW + 16,), jnp.int32),
            pltpu.SemaphoreType.DMA,
            pltpu.SemaphoreType.DMA,
            pltpu.SemaphoreType.DMA,
            pltpu.SemaphoreType.DMA,
        ],
    )(_sc_body)
    out = run(x2, codes, struct_w, abs_emb)
    return out.reshape(b, s, d)


# speedup vs baseline: 3.1394x; 3.1394x over previous
import functools

import jax
import jax.numpy as jnp
from jax import lax
from jax.experimental import pallas as pl
from jax.experimental.pallas import tpu as pltpu
from jax.experimental.pallas import tpu_sc as plsc

_D = 1024
_B = 4
_S = 2048
_NW = 32
_SPW = _S // _NW
_ROWS = 8
_CPB = _SPW // _ROWS
_NCH = _B * _CPB
_NVEC = _D // 16


def _sc_body(x_hbm, codes_hbm, w_hbm, abs_hbm, out_hbm,
             abs_v, in0, in1, ou0, ou1, w_v, codes_v,
             isem0, isem1, osem0, osem1):
    wid = lax.axis_index("s") * 2 + lax.axis_index("c")
    s_base = wid * _SPW
    ins = (in0, in1)
    ous = (ou0, ou1)
    isems = (isem0, isem1)
    osems = (osem0, osem1)

    pltpu.sync_copy(w_hbm, w_v)
    pltpu.sync_copy(abs_hbm.at[pl.ds(s_base, _SPW)], abs_v)
    for bb in range(_B):
        pltpu.sync_copy(codes_hbm.at[pl.ds(bb * _S + s_base, _SPW)],
                        codes_v.at[pl.ds(bb * _SPW, _SPW)])

    def x_base(m):
        return (m // _CPB) * _S + s_base + (m % _CPB) * _ROWS

    for b in range(2):
        pltpu.async_copy(x_hbm.at[pl.ds(x_base(b), _ROWS)], ins[b], isems[b])

    def step(k, _):
        for b in range(2):
            m = 2 * k + b
            pltpu.make_async_copy(
                x_hbm.at[pl.ds(0, _ROWS)], ins[b], isems[b]).wait()
            @pl.when(k > 0)
            def _():
                pltpu.make_async_copy(
                    ous[b], out_hbm.at[pl.ds(0, _ROWS)], osems[b]).wait()

            coff = (m // _CPB) * _SPW + (m % _CPB) * _ROWS
            aoff = (m % _CPB) * _ROWS

            @plsc.parallel_loop(0, _ROWS, 1, unroll=2)
            def _(t, b=b, coff=coff, aoff=aoff):
                cvec = codes_v[pl.ds(coff + t, 16)]
                c = cvec[0]
                for g in range(_NVEC // 4):
                    sls = [pl.ds((4 * g + q) * 16, 16) for q in range(4)]
                    xs = [ins[b][t, sl] for sl in sls]
                    avs = [abs_v[aoff + t, sl] for sl in sls]
                    wvs = [w_v[c, sl] for sl in sls]
                    tmps = [xv + av for xv, av in zip(xs, avs)]
                    for sl, tmp, wv in zip(sls, tmps, wvs):
                        ous[b][t, sl] = tmp + wv
            pltpu.async_copy(ous[b], out_hbm.at[pl.ds(x_base(m), _ROWS)],
                             osems[b])

            @pl.when(k < (_NCH // 2 - 1))
            def _():
                pltpu.async_copy(x_hbm.at[pl.ds(x_base(m + 2), _ROWS)],
                                 ins[b], isems[b])
        return 0

    lax.fori_loop(0, _NCH // 2, step, 0)
    for b in range(2):
        pltpu.make_async_copy(
            ous[b], out_hbm.at[pl.ds(0, _ROWS)], osems[b]).wait()


def kernel(x, pos_codes, struct_w, abs_emb):
    b, s, d = x.shape
    x2 = x.reshape(b * s, d)
    codes = pos_codes.astype(jnp.int32).reshape(b * s)
    mesh = plsc.VectorSubcoreMesh(core_axis_name="c", subcore_axis_name="s")
    run = functools.partial(
        pl.kernel,
        mesh=mesh,
        out_type=jax.ShapeDtypeStruct((b * s, d), jnp.float32),
        scratch_types=[
            pltpu.VMEM((_SPW, _D), jnp.float32),
            pltpu.VMEM((_ROWS, _D), jnp.float32),
            pltpu.VMEM((_ROWS, _D), jnp.float32),
            pltpu.VMEM((_ROWS, _D), jnp.float32),
            pltpu.VMEM((_ROWS, _D), jnp.float32),
            pltpu.VMEM((5, _D), jnp.float32),
            pltpu.VMEM((_B * _SPW + 16,), jnp.int32),
            pltpu.SemaphoreType.DMA,
            pltpu.SemaphoreType.DMA,
            pltpu.SemaphoreType.DMA,
            pltpu.SemaphoreType.DMA,
        ],
    )(_sc_body)
    out = run(x2, codes, struct_w, abs_emb)
    return out.reshape(b, s, d)


# speedup vs baseline: 3.2300x; 1.0289x over previous
import functools

import jax
import jax.numpy as jnp
from jax import lax
from jax.experimental import pallas as pl
from jax.experimental.pallas import tpu as pltpu
from jax.experimental.pallas import tpu_sc as plsc

_D = 1024
_B = 4
_S = 2048
_NW = 32
_SPW = _S // _NW
_ROWS = 8
_CPB = _SPW // _ROWS
_NCH = _B * _CPB
_NVEC = _D // 16


def _sc_body(x_hbm, codes_hbm, w_hbm, abs_hbm, out_hbm,
             abs_v, in0, in1, ou0, ou1, w_v, codes_v,
             isem0, isem1, osem0, osem1):
    wid = lax.axis_index("s") * 2 + lax.axis_index("c")
    s_base = wid * _SPW
    ins = (in0, in1)
    ous = (ou0, ou1)
    isems = (isem0, isem1)
    osems = (osem0, osem1)

    pltpu.sync_copy(w_hbm, w_v)
    pltpu.sync_copy(abs_hbm.at[pl.ds(s_base, _SPW)], abs_v)
    for bb in range(_B):
        pltpu.sync_copy(codes_hbm.at[pl.ds(bb * _S + s_base, _SPW)],
                        codes_v.at[pl.ds(bb * _SPW, _SPW)])

    def x_base(m):
        return (m // _CPB) * _S + s_base + (m % _CPB) * _ROWS

    for b in range(2):
        pltpu.async_copy(x_hbm.at[pl.ds(x_base(b), _ROWS)], ins[b], isems[b])

    def step(k, _):
        for b in range(2):
            m = 2 * k + b
            pltpu.make_async_copy(
                x_hbm.at[pl.ds(0, _ROWS)], ins[b], isems[b]).wait()
            @pl.when(k > 0)
            def _():
                pltpu.make_async_copy(
                    ous[b], out_hbm.at[pl.ds(0, _ROWS)], osems[b]).wait()

            coff = (m // _CPB) * _SPW + (m % _CPB) * _ROWS
            aoff = (m % _CPB) * _ROWS

            @plsc.parallel_loop(0, _ROWS, 1, unroll=2)
            def _(t, b=b, coff=coff, aoff=aoff):
                cvec = codes_v[pl.ds(coff + t, 16)]
                c = cvec[0]
                for g in range(_NVEC // 8):
                    sls = [pl.ds((8 * g + q) * 16, 16) for q in range(8)]
                    xs = [ins[b][t, sl] for sl in sls]
                    avs = [abs_v[aoff + t, sl] for sl in sls]
                    wvs = [w_v[c, sl] for sl in sls]
                    tmps = [xv + av for xv, av in zip(xs, avs)]
                    for sl, tmp, wv in zip(sls, tmps, wvs):
                        ous[b][t, sl] = tmp + wv
            pltpu.async_copy(ous[b], out_hbm.at[pl.ds(x_base(m), _ROWS)],
                             osems[b])

            @pl.when(k < (_NCH // 2 - 1))
            def _():
                pltpu.async_copy(x_hbm.at[pl.ds(x_base(m + 2), _ROWS)],
                                 ins[b], isems[b])
        return 0

    lax.fori_loop(0, _NCH // 2, step, 0)
    for b in range(2):
        pltpu.make_async_copy(
            ous[b], out_hbm.at[pl.ds(0, _ROWS)], osems[b]).wait()


def kernel(x, pos_codes, struct_w, abs_emb):
    b, s, d = x.shape
    x2 = x.reshape(b * s, d)
    codes = pos_codes.astype(jnp.int32).reshape(b * s)
    mesh = plsc.VectorSubcoreMesh(core_axis_name="c", subcore_axis_name="s")
    run = functools.partial(
        pl.kernel,
        mesh=mesh,
        out_type=jax.ShapeDtypeStruct((b * s, d), jnp.float32),
        scratch_types=[
            pltpu.VMEM((_SPW, _D), jnp.float32),
            pltpu.VMEM((_ROWS, _D), jnp.float32),
            pltpu.VMEM((_ROWS, _D), jnp.float32),
            pltpu.VMEM((_ROWS, _D), jnp.float32),
            pltpu.VMEM((_ROWS, _D), jnp.float32),
            pltpu.VMEM((5, _D), jnp.float32),
            pltpu.VMEM((_B * _SPW + 16,), jnp.int32),
            pltpu.SemaphoreType.DMA,
            pltpu.SemaphoreType.DMA,
            pltpu.SemaphoreType.DMA,
            pltpu.SemaphoreType.DMA,
        ],
    )(_sc_body)
    out = run(x2, codes, struct_w, abs_emb)
    return out.reshape(b, s, d)
